# baseline (speedup 1.0000x reference)
import numpy as np
import jax
import jax.numpy as jnp
from jax import lax
from functools import partial
from jax.experimental import pallas as pl
from jax.experimental.pallas import tpu as pltpu
from jax.experimental.pallas import tpu_sc as plsc

HIGHEST = lax.Precision.HIGHEST
F32 = jnp.float32
I32 = jnp.int32

N_POINTS = 128
IMG = 512
NIMG = 24
NPTS = 3072
MAX_STEPS = 10
OOB = np.float32(1e-5)
NOISE_STD = 0.5
NW = 32

_MESH = plsc.VectorSubcoreMesh(core_axis_name="c", subcore_axis_name="s")

DOWN_SIZES = [512, 256, 128, 64, 32, 16, 8, 4, 2]
IPB = 2


def _weight_mat(n, out=IMG):
    scale = out / n
    inv_scale = 1.0 / scale
    sample_f = (jnp.arange(out, dtype=F32) + 0.5) * inv_scale - 0.5
    x = jnp.abs(sample_f[np.newaxis, :] - jnp.arange(n, dtype=F32)[:, np.newaxis])
    weights = jnp.maximum(0, 1 - x)
    total = jnp.sum(weights, axis=0, keepdims=True)
    weights = jnp.where(
        jnp.abs(total) > 1000. * float(np.finfo(np.float32).eps),
        jnp.divide(weights, jnp.where(total != 0, total, 1)), 0)
    weights = jnp.where(
        jnp.logical_and(sample_f >= -0.5, sample_f <= n - 0.5)[np.newaxis, :],
        weights, 0)
    return weights


def _selmat(n):
    E = np.zeros((n, n // 2), np.float32)
    E[np.arange(0, n, 2), np.arange(n // 2)] = 1.0
    return jnp.asarray(E)


def _make_sc_kernels(npts):
    ppw = npts // NW

    @partial(pl.kernel, mesh=_MESH,
             out_type=jax.ShapeDtypeStruct((npts,), I32),
             scratch_types=[pltpu.VMEM((ppw,), F32),
                            pltpu.VMEM((ppw,), F32),
                            pltpu.VMEM((ppw,), I32),
                            pltpu.VMEM((ppw,), I32),
                            pltpu.VMEM((ppw,), F32)])
    def sc_init_scatter(mo_ref, px_hbm, py_hbm, bc_hbm, ones_hbm, idx_hbm,
                        pxv, pyv, bcv, iv, onesv):
        wid = lax.axis_index("s") * 2 + lax.axis_index("c")
        base = wid * ppw
        pltpu.sync_copy(px_hbm.at[pl.ds(base, ppw)], pxv)
        pltpu.sync_copy(py_hbm.at[pl.ds(base, ppw)], pyv)
        pltpu.sync_copy(bc_hbm.at[pl.ds(base, ppw)], bcv)
        pltpu.sync_copy(ones_hbm.at[pl.ds(base, ppw)], onesv)
        for c in range(ppw // 16):
            sl = pl.ds(c * 16, 16)
            xr = (pxv[sl] + 8388608.0) - 8388608.0
            yr = (pyv[sl] + 8388608.0) - 8388608.0
            xi = jnp.clip(xr.astype(I32), 0, IMG - 1)
            yi = jnp.clip(yr.astype(I32), 0, IMG - 1)
            iv[sl] = yi * IMG + xi + bcv[sl]
        pltpu.sync_copy(onesv, mo_ref.at[iv])
        pltpu.sync_copy(iv, idx_hbm.at[pl.ds(base, ppw)])

    @partial(pl.kernel, mesh=_MESH, out_type=(),
             scratch_types=[pltpu.VMEM((ppw,), I32),
                            pltpu.VMEM((ppw,), F32)])
    def sc_scatter(mo_ref, idx_hbm, ones_hbm, iv, onesv):
        wid = lax.axis_index("s") * 2 + lax.axis_index("c")
        base = wid * ppw
        pltpu.sync_copy(idx_hbm.at[pl.ds(base, ppw)], iv)
        pltpu.sync_copy(ones_hbm.at[pl.ds(base, ppw)], onesv)
        pltpu.sync_copy(onesv, mo_ref.at[iv])

    @partial(pl.kernel, mesh=_MESH,
             out_type=jax.ShapeDtypeStruct((3 * npts,), F32),
             scratch_types=[pltpu.VMEM((ppw,), I32),
                            pltpu.VMEM((ppw,), F32),
                            pltpu.VMEM((ppw,), F32),
                            pltpu.VMEM((ppw,), F32),
                            pltpu.SemaphoreType.DMA])
    def sc_gather(j0_hbm, j1_hbm, mt_hbm, idx_hbm, o_hbm,
                  iv, g0v, g1v, g2v, sem):
        wid = lax.axis_index("s") * 2 + lax.axis_index("c")
        base = wid * ppw
        pltpu.sync_copy(idx_hbm.at[pl.ds(base, ppw)], iv)
        c0 = pltpu.async_copy(j0_hbm.at[iv], g0v, sem)
        c1 = pltpu.async_copy(j1_hbm.at[iv], g1v, sem)
        c2 = pltpu.async_copy(mt_hbm.at[iv], g2v, sem)
        c0.wait()
        c1.wait()
        c2.wait()
        pltpu.sync_copy(g0v, o_hbm.at[pl.ds(base, ppw)])
        pltpu.sync_copy(g1v, o_hbm.at[pl.ds(npts + base, ppw)])
        pltpu.sync_copy(g2v, o_hbm.at[pl.ds(2 * npts + base, ppw)])

    return sc_init_scatter, sc_scatter, sc_gather


def _level_down(x, n, E):
    il = lax.broadcasted_iota(I32, (n, n), 1)
    isr = lax.broadcasted_iota(I32, (n, n), 0)
    left = jnp.where(il == 0, OOB, pltpu.roll(x, 1, 1))
    right = jnp.where(il == n - 1, OOB, pltpu.roll(x, n - 1, 1))
    s = (left + x) + right
    pc = np.float32(np.float32(OOB + OOB) + OOB)
    up = jnp.where(isr == 0, pc, pltpu.roll(s, 1, 0))
    down = jnp.where(isr == n - 1, pc, pltpu.roll(s, n - 1, 0))
    t = (up + s) + down
    dec = lax.dot_general(E, t, (((0,), (0,)), ((), ())),
                          precision=HIGHEST, preferred_element_type=F32)
    dec = lax.dot_general(dec, E, (((1,), (0,)), ((), ())),
                          precision=HIGHEST, preferred_element_type=F32)
    return dec / 9.0


def _j_body(mt_ref, mo_ref, *refs):
    e_refs = refs[:9]
    w_refs = refs[9:17]
    j0_ref, j1_ref = refs[17], refs[18]
    il = lax.broadcasted_iota(I32, (IMG, IMG), 1)
    isr = lax.broadcasted_iota(I32, (IMG, IMG), 0)
    for k in range(IPB):
        g = 0.01 * mo_ref[k] - mt_ref[k]
        gsum = g
        cur = g
        for li, n in enumerate(DOWN_SIZES):
            cur = _level_down(cur, n, e_refs[li][...])
            m = n // 2
            if m > 1:
                w = w_refs[li][...]
                t1 = lax.dot_general(cur, w, (((0,), (0,)), ((), ())),
                                     precision=HIGHEST, preferred_element_type=F32)
                u = lax.dot_general(t1, w, (((0,), (0,)), ((), ())),
                                    precision=HIGHEST, preferred_element_type=F32)
                gsum = gsum + u
            else:
                gsum = gsum + cur[0, 0]
        ga = gsum / 10.0
        upe = jnp.where(isr == 0, ga, pltpu.roll(ga, 1, 0))
        dne = jnp.where(isr == IMG - 1, ga, pltpu.roll(ga, IMG - 1, 0))
        j0_ref[k] = (upe - dne) / 2.0
        lfe = jnp.where(il == 0, ga, pltpu.roll(ga, 1, 1))
        rte = jnp.where(il == IMG - 1, ga, pltpu.roll(ga, IMG - 1, 1))
        j1_ref[k] = (lfe - rte) / 2.0


def _run_j(mt24, mo24, e_mats, w_mats):
    def full(shp):
        nd = len(shp)
        return pl.BlockSpec(shp, lambda i, _nd=nd: (0,) * _nd)
    nimg = mt24.shape[0]
    in_specs = [pl.BlockSpec((IPB, IMG, IMG), lambda i: (i, 0, 0)),
                pl.BlockSpec((IPB, IMG, IMG), lambda i: (i, 0, 0))]
    in_specs += [full(tuple(e.shape)) for e in e_mats]
    in_specs += [full(tuple(w.shape)) for w in w_mats]
    return pl.pallas_call(
        _j_body,
        grid=(nimg // IPB,),
        in_specs=in_specs,
        out_specs=[pl.BlockSpec((IPB, IMG, IMG), lambda i: (i, 0, 0)),
                   pl.BlockSpec((IPB, IMG, IMG), lambda i: (i, 0, 0))],
        out_shape=[jax.ShapeDtypeStruct((nimg, IMG, IMG), F32),
                   jax.ShapeDtypeStruct((nimg, IMG, IMG), F32)],
    )(mt24, mo24, *e_mats, *w_mats)


def _step_body(first, g_ref, p_ref, sq_ref, act_ref, nx_ref, ny_ref,
               dec_ref, bc_ref, po_ref, sqo_ref, acto_ref, idxo_ref):
    npts = dec_ref.shape[0]
    mtg = g_ref[2]
    plv = (1.0 - mtg) * dec_ref[...]
    gate = jnp.max(plv) > 0.1
    if first:
        act = jnp.broadcast_to(gate, (npts,))
    else:
        act = jnp.logical_and(act_ref[...] > 0.5, gate)
    actf = act.astype(F32)

    gdx = g_ref[1] + NOISE_STD * nx_ref[...]
    gdy = g_ref[0] + NOISE_STD * ny_ref[...]
    px, py = p_ref[0], p_ref[1]
    sqx, sqy = sq_ref[0], sq_ref[1]

    gradx = jnp.negative(gdx) * plv
    grady = jnp.negative(gdy) * plv
    nsqx = 0.99 * sqx + 0.01 * gradx * gradx
    nsqy = 0.99 * sqy + 0.01 * grady * grady
    npx = px - 0.2 * gradx / (jnp.sqrt(nsqx) + 1e-08)
    npy = py - 0.2 * grady / (jnp.sqrt(nsqy) + 1e-08)
    px2 = jnp.where(act, npx, px)
    py2 = jnp.where(act, npy, py)
    po_ref[0] = px2
    po_ref[1] = py2
    sqo_ref[0] = jnp.where(act, nsqx, sqx)
    sqo_ref[1] = jnp.where(act, nsqy, sqy)
    acto_ref[...] = actf

    xi = jnp.clip(jnp.round(px2).astype(I32), 0, IMG - 1)
    yi = jnp.clip(jnp.round(py2).astype(I32), 0, IMG - 1)
    idxo_ref[...] = yi * IMG + xi + bc_ref[...]


def _run_step(first, g3, p, sq, act, nx, ny, decay_t, bc_off):
    npts = decay_t.shape[0]
    return pl.pallas_call(
        partial(_step_body, first),
        out_shape=[jax.ShapeDtypeStruct((2, npts), F32),
                   jax.ShapeDtypeStruct((2, npts), F32),
                   jax.ShapeDtypeStruct((npts,), F32),
                   jax.ShapeDtypeStruct((npts,), I32)],
    )(g3, p, sq, act, nx, ny, decay_t, bc_off)


def _core(nd, mt24, px0_l, py0_l, pxf, pyf, decay_t, nxs, nys, ones_f):
    npts_l = px0_l.shape[0]
    nimg_l = mt24.shape[0]
    npix_l = nimg_l * IMG * IMG
    npts_f = pxf.shape[0]
    sc_init, sc_scat, _ = _make_sc_kernels(npts_l)
    _, _, sc_gath = _make_sc_kernels(npts_f)
    bc_l = (jnp.arange(npts_l, dtype=I32) // N_POINTS) * (IMG * IMG)
    bc_f = (jnp.arange(npts_f, dtype=I32) // N_POINTS) * (IMG * IMG)
    ones_l = ones_f[:npts_l]

    e_mats = [_selmat(n) for n in DOWN_SIZES]
    w_mats = [_weight_mat(n // 2) for n in DOWN_SIZES if n // 2 > 1]

    mo_ref = jax.new_ref(jnp.zeros((npix_l,), F32))
    idx_l = sc_init(mo_ref, px0_l, py0_l, bc_l, ones_l)
    masks_out_flat = mo_ref[...]

    j0, j1 = _run_j(mt24, masks_out_flat.reshape(nimg_l, IMG, IMG),
                    e_mats, w_mats)

    if nd > 1:
        shard_off = jnp.arange(nd, dtype=I32)[:, None] * npix_l
        idx_f = (lax.all_gather(idx_l, 'd') + shard_off).reshape(npts_f)
        j0f = lax.all_gather(j0, 'd', axis=0, tiled=True).reshape(nd * npix_l)
        j1f = lax.all_gather(j1, 'd', axis=0, tiled=True).reshape(nd * npix_l)
        mtf = lax.all_gather(mt24, 'd', axis=0, tiled=True).reshape(nd * npix_l)
    else:
        idx_f = idx_l
        j0f = j0.reshape(npix_l)
        j1f = j1.reshape(npix_l)
        mtf = mt24.reshape(npix_l)

    p = jnp.stack([pxf, pyf])
    sq = jnp.zeros((2, npts_f), F32)
    act = ones_f
    idx = idx_f
    for i in range(MAX_STEPS):
        g3 = sc_gath(j0f, j1f, mtf, idx).reshape(3, npts_f)
        p, sq, act, idx = _run_step(i == 0, g3, p, sq, act,
                                    nxs[i], nys[i], decay_t, bc_f)

    if nd > 1:
        didx = lax.axis_index('d')
        idx_loc = lax.dynamic_slice(idx, (didx * npts_l,), (npts_l,)) \
            - didx * npix_l
    else:
        idx_loc = idx
    mopt_ref = jax.new_ref(jnp.zeros((npix_l,), F32))
    sc_scat(mopt_ref, idx_loc, ones_l)
    masks_opt_flat = mopt_ref[...]
    return masks_out_flat, masks_opt_flat, p


def kernel(points_2d, masks_target):
    from jax.sharding import Mesh, PartitionSpec as P

    mt24 = masks_target.reshape(NIMG, IMG, IMG)
    px0 = points_2d[..., 0].reshape(NPTS)
    py0 = points_2d[..., 1].reshape(NPTS)
    ones = jnp.ones((NPTS,), F32)

    decay = jnp.exp(-jnp.arange(N_POINTS, dtype=F32) / N_POINTS * 10.0)
    decay_t = jnp.tile(decay, NIMG)

    noise_key = jax.random.key(1234)
    nxs, nys = [], []
    for i in range(MAX_STEPS):
        n = jax.random.normal(jax.random.fold_in(noise_key, i),
                              (8, 3, N_POINTS, 2), dtype=F32)
        nxs.append(n[..., 0].reshape(NPTS))
        nys.append(n[..., 1].reshape(NPTS))
    nxs = jnp.stack(nxs)
    nys = jnp.stack(nys)

    devs = jax.devices()
    nd = 2 if len(devs) >= 2 else 1
    mesh = Mesh(np.asarray(devs[:nd]), ('d',))
    mo_flat, mopt_flat, p = jax.shard_map(
        partial(_core, nd), mesh=mesh,
        in_specs=(P('d'), P('d'), P('d'), P(), P(), P(), P(), P(), P()),
        out_specs=(P('d'), P('d'), P()),
        check_vma=False,
    )(mt24, px0, py0, px0, py0, decay_t, nxs, nys, ones)

    masks_out = mo_flat.reshape(8, 3, IMG, IMG)
    masks_opt = mopt_flat.reshape(8, 3, IMG, IMG)
    p_opt = jnp.stack([p[0], p[1]], axis=-1).reshape(8, 3, N_POINTS, 2)
    return (masks_out, masks_opt, p_opt)


# speedup vs baseline: 1.5229x; 1.5229x over previous
import numpy as np
import jax
import jax.numpy as jnp
from jax import lax
from functools import partial
from jax.experimental import pallas as pl
from jax.experimental.pallas import tpu as pltpu
from jax.experimental.pallas import tpu_sc as plsc

HIGHEST = lax.Precision.HIGHEST
F32 = jnp.float32
I32 = jnp.int32

N_POINTS = 128
IMG = 512
NIMG = 24
NPTS = 3072
NPIX = NIMG * IMG * IMG
MAX_STEPS = 10
OOB = np.float32(1e-5)
NOISE_STD = 0.5
NW = 32
PPW = NPTS // NW

_MESH = plsc.VectorSubcoreMesh(core_axis_name="c", subcore_axis_name="s")

DOWN_SIZES = [512, 256, 128, 64, 32, 16, 8, 4, 2]


def _weight_mat(n, out=IMG):
    scale = out / n
    inv_scale = 1.0 / scale
    sample_f = (jnp.arange(out, dtype=F32) + 0.5) * inv_scale - 0.5
    x = jnp.abs(sample_f[np.newaxis, :] - jnp.arange(n, dtype=F32)[:, np.newaxis])
    weights = jnp.maximum(0, 1 - x)
    total = jnp.sum(weights, axis=0, keepdims=True)
    weights = jnp.where(
        jnp.abs(total) > 1000. * float(np.finfo(np.float32).eps),
        jnp.divide(weights, jnp.where(total != 0, total, 1)), 0)
    weights = jnp.where(
        jnp.logical_and(sample_f >= -0.5, sample_f <= n - 0.5)[np.newaxis, :],
        weights, 0)
    return weights


def _selmat(n):
    E = np.zeros((n, n // 2), np.float32)
    E[np.arange(0, n, 2), np.arange(n // 2)] = 1.0
    return jnp.asarray(E)


@partial(pl.kernel, mesh=_MESH,
         out_type=jax.ShapeDtypeStruct((NPTS,), I32),
         scratch_types=[pltpu.VMEM((PPW,), F32),
                        pltpu.VMEM((PPW,), F32),
                        pltpu.VMEM((PPW,), I32),
                        pltpu.VMEM((PPW,), I32),
                        pltpu.VMEM((PPW,), F32)])
def _sc_init_scatter(mo_ref, px_hbm, py_hbm, bc_hbm, ones_hbm, idx_hbm,
                     pxv, pyv, bcv, iv, onesv):
    wid = lax.axis_index("s") * 2 + lax.axis_index("c")
    base = wid * PPW
    pltpu.sync_copy(px_hbm.at[pl.ds(base, PPW)], pxv)
    pltpu.sync_copy(py_hbm.at[pl.ds(base, PPW)], pyv)
    pltpu.sync_copy(bc_hbm.at[pl.ds(base, PPW)], bcv)
    pltpu.sync_copy(ones_hbm.at[pl.ds(base, PPW)], onesv)
    for c in range(PPW // 16):
        sl = pl.ds(c * 16, 16)
        xr = (pxv[sl] + 8388608.0) - 8388608.0
        yr = (pyv[sl] + 8388608.0) - 8388608.0
        xi = jnp.clip(xr.astype(I32), 0, IMG - 1)
        yi = jnp.clip(yr.astype(I32), 0, IMG - 1)
        iv[sl] = yi * IMG + xi + bcv[sl]
    pltpu.sync_copy(onesv, mo_ref.at[iv])
    pltpu.sync_copy(iv, idx_hbm.at[pl.ds(base, PPW)])


@partial(pl.kernel, mesh=_MESH, out_type=(),
         scratch_types=[pltpu.VMEM((PPW,), I32),
                        pltpu.VMEM((PPW,), F32)])
def _sc_scatter(mo_ref, idx_hbm, ones_hbm, iv, onesv):
    wid = lax.axis_index("s") * 2 + lax.axis_index("c")
    base = wid * PPW
    pltpu.sync_copy(idx_hbm.at[pl.ds(base, PPW)], iv)
    pltpu.sync_copy(ones_hbm.at[pl.ds(base, PPW)], onesv)
    pltpu.sync_copy(onesv, mo_ref.at[iv])


@partial(pl.kernel, mesh=_MESH,
         out_type=jax.ShapeDtypeStruct((3 * NPTS,), F32),
         scratch_types=[pltpu.VMEM((PPW,), I32),
                        pltpu.VMEM((PPW,), F32),
                        pltpu.VMEM((PPW,), F32),
                        pltpu.VMEM((PPW,), F32),
                        pltpu.SemaphoreType.DMA])
def _sc_gather(j0_hbm, j1_hbm, mt_hbm, idx_hbm, o_hbm, iv, g0v, g1v, g2v, sem):
    wid = lax.axis_index("s") * 2 + lax.axis_index("c")
    base = wid * PPW
    pltpu.sync_copy(idx_hbm.at[pl.ds(base, PPW)], iv)
    c0 = pltpu.async_copy(j0_hbm.at[iv], g0v, sem)
    c1 = pltpu.async_copy(j1_hbm.at[iv], g1v, sem)
    c2 = pltpu.async_copy(mt_hbm.at[iv], g2v, sem)
    c0.wait()
    c1.wait()
    c2.wait()
    pltpu.sync_copy(g0v, o_hbm.at[pl.ds(base, PPW)])
    pltpu.sync_copy(g1v, o_hbm.at[pl.ds(NPTS + base, PPW)])
    pltpu.sync_copy(g2v, o_hbm.at[pl.ds(2 * NPTS + base, PPW)])


def _level_down(x, n, E):
    il = lax.broadcasted_iota(I32, (n, n), 1)
    isr = lax.broadcasted_iota(I32, (n, n), 0)
    left = jnp.where(il == 0, OOB, pltpu.roll(x, 1, 1))
    right = jnp.where(il == n - 1, OOB, pltpu.roll(x, n - 1, 1))
    s = (left + x) + right
    pc = np.float32(np.float32(OOB + OOB) + OOB)
    up = jnp.where(isr == 0, pc, pltpu.roll(s, 1, 0))
    down = jnp.where(isr == n - 1, pc, pltpu.roll(s, n - 1, 0))
    t = (up + s) + down
    dec = lax.dot_general(E, t, (((0,), (0,)), ((), ())),
                          precision=HIGHEST, preferred_element_type=F32)
    dec = lax.dot_general(dec, E, (((1,), (0,)), ((), ())),
                          precision=HIGHEST, preferred_element_type=F32)
    return dec / 9.0


IPB = 2


def _j_body(mt_ref, mo_ref, *refs):
    e_refs = refs[:9]
    w_refs = refs[9:17]
    j0_ref, j1_ref = refs[17], refs[18]
    il = lax.broadcasted_iota(I32, (IMG, IMG), 1)
    isr = lax.broadcasted_iota(I32, (IMG, IMG), 0)
    for k in range(IPB):
        g = 0.01 * mo_ref[k] - mt_ref[k]
        gsum = g
        cur = g
        for li, n in enumerate(DOWN_SIZES):
            cur = _level_down(cur, n, e_refs[li][...])
            m = n // 2
            if m > 1:
                w = w_refs[li][...]
                t1 = lax.dot_general(cur, w, (((0,), (0,)), ((), ())),
                                     precision=HIGHEST, preferred_element_type=F32)
                u = lax.dot_general(t1, w, (((0,), (0,)), ((), ())),
                                    precision=HIGHEST, preferred_element_type=F32)
                gsum = gsum + u
            else:
                gsum = gsum + cur[0, 0]
        ga = gsum / 10.0
        upe = jnp.where(isr == 0, ga, pltpu.roll(ga, 1, 0))
        dne = jnp.where(isr == IMG - 1, ga, pltpu.roll(ga, IMG - 1, 0))
        j0_ref[k] = (upe - dne) / 2.0
        lfe = jnp.where(il == 0, ga, pltpu.roll(ga, 1, 1))
        rte = jnp.where(il == IMG - 1, ga, pltpu.roll(ga, IMG - 1, 1))
        j1_ref[k] = (lfe - rte) / 2.0


def _run_j(mt24, mo24, e_mats, w_mats):
    def full(shp):
        nd = len(shp)
        return pl.BlockSpec(shp, lambda i, _nd=nd: (0,) * _nd)
    in_specs = [pl.BlockSpec((IPB, IMG, IMG), lambda i: (i, 0, 0)),
                pl.BlockSpec((IPB, IMG, IMG), lambda i: (i, 0, 0))]
    in_specs += [full(tuple(e.shape)) for e in e_mats]
    in_specs += [full(tuple(w.shape)) for w in w_mats]
    return pl.pallas_call(
        _j_body,
        grid=(NIMG // IPB,),
        in_specs=in_specs,
        out_specs=[pl.BlockSpec((IPB, IMG, IMG), lambda i: (i, 0, 0)),
                   pl.BlockSpec((IPB, IMG, IMG), lambda i: (i, 0, 0))],
        out_shape=[jax.ShapeDtypeStruct((NIMG, IMG, IMG), F32),
                   jax.ShapeDtypeStruct((NIMG, IMG, IMG), F32)],
    )(mt24, mo24, *e_mats, *w_mats)


def _step_body(first, g_ref, p_ref, sq_ref, act_ref, nx_ref, ny_ref,
               dec_ref, bc_ref, po_ref, sqo_ref, acto_ref, idxo_ref):
    mtg = g_ref[2]
    plv = (1.0 - mtg) * dec_ref[...]
    mx = jnp.max(plv)
    gate = mx > 0.1
    if first:
        act = jnp.broadcast_to(gate, (NPTS,))
    else:
        act = jnp.logical_and(act_ref[...] > 0.5, gate)
    actf = act.astype(F32)

    gdx = g_ref[1] + NOISE_STD * nx_ref[...]
    gdy = g_ref[0] + NOISE_STD * ny_ref[...]
    px, py = p_ref[0], p_ref[1]
    sqx, sqy = sq_ref[0], sq_ref[1]

    gradx = jnp.negative(gdx) * plv
    grady = jnp.negative(gdy) * plv
    nsqx = 0.99 * sqx + 0.01 * gradx * gradx
    nsqy = 0.99 * sqy + 0.01 * grady * grady
    npx = px - 0.2 * gradx / (jnp.sqrt(nsqx) + 1e-08)
    npy = py - 0.2 * grady / (jnp.sqrt(nsqy) + 1e-08)
    px2 = jnp.where(act, npx, px)
    py2 = jnp.where(act, npy, py)
    po_ref[0] = px2
    po_ref[1] = py2
    sqo_ref[0] = jnp.where(act, nsqx, sqx)
    sqo_ref[1] = jnp.where(act, nsqy, sqy)
    acto_ref[...] = actf

    xi = jnp.clip(jnp.round(px2).astype(I32), 0, IMG - 1)
    yi = jnp.clip(jnp.round(py2).astype(I32), 0, IMG - 1)
    idxo_ref[...] = yi * IMG + xi + bc_ref[...]


def _run_step(first, g3, p, sq, act, nx, ny, decay_t, bc_off):
    return pl.pallas_call(
        partial(_step_body, first),
        out_shape=[jax.ShapeDtypeStruct((2, NPTS), F32),
                   jax.ShapeDtypeStruct((2, NPTS), F32),
                   jax.ShapeDtypeStruct((NPTS,), F32),
                   jax.ShapeDtypeStruct((NPTS,), I32)],
    )(g3, p, sq, act, nx, ny, decay_t, bc_off)


def kernel(points_2d, masks_target):
    mt24 = masks_target.reshape(NIMG, IMG, IMG)
    mtf = masks_target.reshape(NPIX)
    px0 = points_2d[..., 0].reshape(NPTS)
    py0 = points_2d[..., 1].reshape(NPTS)
    bc_off = (jnp.arange(NPTS, dtype=I32) // N_POINTS) * (IMG * IMG)
    ones = jnp.ones((NPTS,), F32)

    decay = jnp.exp(-jnp.arange(N_POINTS, dtype=F32) / N_POINTS * 10.0)
    decay_t = jnp.tile(decay, NIMG)

    noise_key = jax.random.key(1234)
    nxs, nys = [], []
    for i in range(MAX_STEPS):
        n = jax.random.normal(jax.random.fold_in(noise_key, i),
                              (8, 3, N_POINTS, 2), dtype=F32)
        nxs.append(n[..., 0].reshape(NPTS))
        nys.append(n[..., 1].reshape(NPTS))

    e_mats = [_selmat(n) for n in DOWN_SIZES]
    w_mats = [_weight_mat(n // 2) for n in DOWN_SIZES if n // 2 > 1]

    mo_ref = jax.new_ref(jnp.zeros((NPIX,), F32))
    idx = _sc_init_scatter(mo_ref, px0, py0, bc_off, ones)
    masks_out_flat = mo_ref[...]

    j0, j1 = _run_j(mt24, masks_out_flat.reshape(NIMG, IMG, IMG),
                    e_mats, w_mats)
    j0f = j0.reshape(NPIX)
    j1f = j1.reshape(NPIX)

    p = jnp.stack([px0, py0])
    sq = jnp.zeros((2, NPTS), F32)
    act = ones
    for i in range(MAX_STEPS):
        g3 = _sc_gather(j0f, j1f, mtf, idx).reshape(3, NPTS)
        p, sq, act, idx = _run_step(i == 0, g3, p, sq, act,
                                    nxs[i], nys[i], decay_t, bc_off)

    mopt_ref = jax.new_ref(jnp.zeros((NPIX,), F32))
    _sc_scatter(mopt_ref, idx, ones)
    masks_opt_flat = mopt_ref[...]

    masks_out = masks_out_flat.reshape(8, 3, IMG, IMG)
    masks_opt = masks_opt_flat.reshape(8, 3, IMG, IMG)
    p_opt = jnp.stack([p[0], p[1]], axis=-1).reshape(8, 3, N_POINTS, 2)
    return (masks_out, masks_opt, p_opt)
